# baseline (speedup 1.0000x reference)
import functools

import jax
import jax.numpy as jnp
from jax import lax
from jax.experimental import pallas as pl
from jax.experimental.pallas import tpu as pltpu
from jax.experimental.pallas import tpu_sc as plsc

NC = 2
NS = 16
NW = NC * NS
L = 16

N = 100000
E = 6400000
EPW = E // NW
C = 2000
NCH = EPW // C
VPC = C // L

_MAGIC = 0x5F3759DF
_HI16 = -65536
_A = 1.00094
_C1 = 1.5 * _A
_CH = 0.5 * _A


def _make_sc_stress(H):
  EPWH = H // NW
  NCHH = EPWH // C

  def _sc_body(tab_hbm, eflat_hbm, invd_hbm, out_hbm,
                 tab_v, i0a_v, i1a_v, da_v, i0b_v, i1b_v, db_v,
                 acc_v, tsem, sem_a, sem_b):
      cid = lax.axis_index("c")
      sid = lax.axis_index("s")
      wid = sid * NC + cid
      base0 = wid * EPWH

      tab_cp = pltpu.make_async_copy(tab_hbm, tab_v, tsem)
      tab_cp.start()

      sems = (sem_a, sem_b)
      bufs = ((i0a_v, i1a_v, da_v), (i0b_v, i1b_v, db_v))

      def start(ci, slot):
          base = pl.multiple_of(base0 + ci * C, 16)
          b0, b1, bd = bufs[slot]
          sem = sems[slot]
          pltpu.async_copy(eflat_hbm.at[pl.ds(base, C)], b0, sem)
          pltpu.async_copy(eflat_hbm.at[pl.ds(H + base, C)], b1, sem)
          pltpu.async_copy(invd_hbm.at[pl.ds(base, C)], bd, sem)

      def wait(slot):
          b0, b1, bd = bufs[slot]
          sem = sems[slot]
          pltpu.make_async_copy(eflat_hbm.at[pl.ds(0, C)], b0, sem).wait()
          pltpu.make_async_copy(eflat_hbm.at[pl.ds(0, C)], b1, sem).wait()
          pltpu.make_async_copy(invd_hbm.at[pl.ds(0, C)], bd, sem).wait()

      def compute(slot, acc):
          b0, b1, bd = bufs[slot]

          @plsc.parallel_loop(0, C, step=L, unroll=4, carry=acc)
          def vec_body(o, acc):
              o = pl.multiple_of(o, L)
              idx0 = b0[pl.ds(o, L)]
              idx1 = b1[pl.ds(o, L)]
              p0 = plsc.load_gather(tab_v, [idx0])
              p1 = plsc.load_gather(tab_v, [idx1])
              w = bd[pl.ds(o, L)]
              x0 = plsc.bitcast(p0 << 16, jnp.float32)
              y0 = plsc.bitcast(p0 & _HI16, jnp.float32)
              x1 = plsc.bitcast(p1 << 16, jnp.float32)
              y1 = plsc.bitcast(p1 & _HI16, jnp.float32)
              dx = x0 - x1
              dy = y0 - y1
              s = dx * dx + dy * dy
              r = plsc.bitcast(_MAGIC - (plsc.bitcast(s, jnp.int32) >> 1),
                               jnp.float32)
              h = _CH * s
              r = r * (_C1 - h * r * r)
              eu = s * r
              q = eu * w - 1.0
              return acc + q * q

          return vec_body

      start(0, 0)
      tab_cp.wait()

      def outer(cc, acc):
          ci0 = cc * 2

          start(ci0 + 1, 1)
          wait(0)
          acc = compute(0, acc)

          @pl.when(cc + 1 < NCHH // 2)
          def _():
              start(ci0 + 2, 0)

          wait(1)
          acc = compute(1, acc)
          return acc

      acc = lax.fori_loop(0, NCHH // 2, outer, jnp.zeros((L,), jnp.float32))
      acc_v[...] = acc
      pltpu.sync_copy(acc_v, out_hbm.at[wid])


  return pl.kernel(
      _sc_body,
      out_type=jax.ShapeDtypeStruct((NW, L), jnp.float32),
      mesh=plsc.VectorSubcoreMesh(
          core_axis_name="c", subcore_axis_name="s",
          num_cores=NC, num_subcores=NS),
      compiler_params=pltpu.CompilerParams(needs_layout_passes=False),
      scratch_types=[
          pltpu.VMEM((N,), jnp.int32),
          pltpu.VMEM((C,), jnp.int32),
          pltpu.VMEM((C,), jnp.int32),
          pltpu.VMEM((C,), jnp.float32),
          pltpu.VMEM((C,), jnp.int32),
          pltpu.VMEM((C,), jnp.int32),
          pltpu.VMEM((C,), jnp.float32),
          pltpu.VMEM((L,), jnp.float32),
          pltpu.SemaphoreType.DMA,
          pltpu.SemaphoreType.DMA,
          pltpu.SemaphoreType.DMA,
      ],
  )


_HALF = E // 2
_sc_stress_half = _make_sc_stress(_HALF)


def kernel(node_pos, edge_index, edge_attr):
    nb = node_pos.astype(jnp.bfloat16)
    bits = lax.bitcast_convert_type(nb, jnp.uint16).astype(jnp.uint32)
    packed = lax.bitcast_convert_type(bits[:, 0] | (bits[:, 1] << 16),
                                      jnp.int32)
    e1 = edge_index[:, :_HALF].reshape(-1)
    e2 = edge_index[:, _HALF:].reshape(-1)
    inv_d1 = 1.0 / edge_attr[:_HALF, 0]
    inv_d2 = 1.0 / edge_attr[_HALF:, 0]
    p1 = _sc_stress_half(packed, e1, inv_d1)
    p2 = _sc_stress_half(packed, e2, inv_d2)
    return jnp.sum(p1) + jnp.sum(p2)


# speedup vs baseline: 1.1608x; 1.1608x over previous
import functools

import jax
import jax.numpy as jnp
from jax import lax
from jax.experimental import pallas as pl
from jax.experimental.pallas import tpu as pltpu
from jax.experimental.pallas import tpu_sc as plsc

NC = 2
NS = 16
NW = NC * NS
L = 16

N = 100000
E = 6400000
EPW = E // NW
C = 2000
NCH = EPW // C
NCHH = NCH // 2
VPC = C // L

_MAGIC = 0x5F3759DF
_HI16 = -65536
_A = 1.00094
_C1 = 1.5 * _A
_CH = 0.5 * _A


def _make_sc_stress(half):
  ch0 = half * NCHH

  def _sc_body(tab_hbm, eflat_hbm, invd_hbm, out_hbm,
                 tab_v, i0a_v, i1a_v, da_v, i0b_v, i1b_v, db_v,
                 acc_v, tsem, sem_a, sem_b):
      cid = lax.axis_index("c")
      sid = lax.axis_index("s")
      wid = sid * NC + cid
      base0 = wid * EPW + ch0 * C

      tab_cp = pltpu.make_async_copy(tab_hbm, tab_v, tsem)
      tab_cp.start()

      sems = (sem_a, sem_b)
      bufs = ((i0a_v, i1a_v, da_v), (i0b_v, i1b_v, db_v))

      def start(ci, slot):
          base = pl.multiple_of(base0 + ci * C, 16)
          b0, b1, bd = bufs[slot]
          sem = sems[slot]
          pltpu.async_copy(eflat_hbm.at[pl.ds(base, C)], b0, sem)
          pltpu.async_copy(eflat_hbm.at[pl.ds(E + base, C)], b1, sem)
          pltpu.async_copy(invd_hbm.at[pl.ds(base, C)], bd, sem)

      def wait(slot):
          b0, b1, bd = bufs[slot]
          sem = sems[slot]
          pltpu.make_async_copy(eflat_hbm.at[pl.ds(0, C)], b0, sem).wait()
          pltpu.make_async_copy(eflat_hbm.at[pl.ds(0, C)], b1, sem).wait()
          pltpu.make_async_copy(invd_hbm.at[pl.ds(0, C)], bd, sem).wait()

      def compute(slot, acc):
          b0, b1, bd = bufs[slot]

          @plsc.parallel_loop(0, C, step=L, unroll=4, carry=acc)
          def vec_body(o, acc):
              o = pl.multiple_of(o, L)
              idx0 = b0[pl.ds(o, L)]
              idx1 = b1[pl.ds(o, L)]
              p0 = plsc.load_gather(tab_v, [idx0])
              p1 = plsc.load_gather(tab_v, [idx1])
              w = bd[pl.ds(o, L)]
              x0 = plsc.bitcast(p0 << 16, jnp.float32)
              y0 = plsc.bitcast(p0 & _HI16, jnp.float32)
              x1 = plsc.bitcast(p1 << 16, jnp.float32)
              y1 = plsc.bitcast(p1 & _HI16, jnp.float32)
              dx = x0 - x1
              dy = y0 - y1
              s = dx * dx + dy * dy
              r = plsc.bitcast(_MAGIC - (plsc.bitcast(s, jnp.int32) >> 1),
                               jnp.float32)
              h = _CH * s
              r = r * (_C1 - h * r * r)
              eu = s * r
              q = eu * w - 1.0
              return acc + q * q

          return vec_body

      start(0, 0)
      tab_cp.wait()

      def outer(cc, acc):
          ci0 = cc * 2

          start(ci0 + 1, 1)
          wait(0)
          acc = compute(0, acc)

          @pl.when(cc + 1 < NCHH // 2)
          def _():
              start(ci0 + 2, 0)

          wait(1)
          acc = compute(1, acc)
          return acc

      acc = lax.fori_loop(0, NCHH // 2, outer, jnp.zeros((L,), jnp.float32))
      acc_v[...] = acc
      pltpu.sync_copy(acc_v, out_hbm.at[wid])


  return pl.kernel(
      _sc_body,
      out_type=jax.ShapeDtypeStruct((NW, L), jnp.float32),
      mesh=plsc.VectorSubcoreMesh(
          core_axis_name="c", subcore_axis_name="s",
          num_cores=NC, num_subcores=NS),
      compiler_params=pltpu.CompilerParams(needs_layout_passes=False),
      scratch_types=[
          pltpu.VMEM((N,), jnp.int32),
          pltpu.VMEM((C,), jnp.int32),
          pltpu.VMEM((C,), jnp.int32),
          pltpu.VMEM((C,), jnp.float32),
          pltpu.VMEM((C,), jnp.int32),
          pltpu.VMEM((C,), jnp.int32),
          pltpu.VMEM((C,), jnp.float32),
          pltpu.VMEM((L,), jnp.float32),
          pltpu.SemaphoreType.DMA,
          pltpu.SemaphoreType.DMA,
          pltpu.SemaphoreType.DMA,
      ],
  )


_sc_stress_h0 = _make_sc_stress(0)
_sc_stress_h1 = _make_sc_stress(1)


def kernel(node_pos, edge_index, edge_attr):
    nb = node_pos.astype(jnp.bfloat16)
    bits = lax.bitcast_convert_type(nb, jnp.uint16).astype(jnp.uint32)
    packed = lax.bitcast_convert_type(bits[:, 0] | (bits[:, 1] << 16),
                                      jnp.int32)
    eflat = edge_index.reshape(-1)
    inv_d = 1.0 / edge_attr[:, 0]
    p0 = _sc_stress_h0(packed, eflat, inv_d)
    p1 = _sc_stress_h1(packed, eflat, inv_d)
    return jnp.sum(p0) + jnp.sum(p1)


# speedup vs baseline: 1.3619x; 1.1732x over previous
import functools

import jax
import jax.numpy as jnp
from jax import lax
from jax.experimental import pallas as pl
from jax.experimental.pallas import tpu as pltpu
from jax.experimental.pallas import tpu_sc as plsc

NC = 2
NS = 16
NW = NC * NS
L = 16

N = 100000
E = 6400000
EPW = E // NW
C = 4000
NCH = EPW // C
VPC = C // L

_MAGIC = 0x5F3759DF
_HI16 = -65536
_A = 1.00094
_C1 = 1.5 * _A
_CH = 0.5 * _A


def _sc_body(tab_hbm, eflat_hbm, invd_hbm, out_hbm,
             tab_v, i0a_v, i1a_v, da_v, i0b_v, i1b_v, db_v,
             acc_v, tsem, sem_a, sem_b):
    cid = lax.axis_index("c")
    sid = lax.axis_index("s")
    wid = sid * NC + cid
    base0 = wid * EPW

    tab_cp = pltpu.make_async_copy(tab_hbm, tab_v, tsem)
    tab_cp.start()

    sems = (sem_a, sem_b)
    bufs = ((i0a_v, i1a_v, da_v), (i0b_v, i1b_v, db_v))

    def start(ci, slot):
        base = pl.multiple_of(base0 + ci * C, 16)
        b0, b1, bd = bufs[slot]
        sem = sems[slot]
        pltpu.async_copy(eflat_hbm.at[pl.ds(base, C)], b0, sem)
        pltpu.async_copy(eflat_hbm.at[pl.ds(E + base, C)], b1, sem)
        pltpu.async_copy(invd_hbm.at[pl.ds(base, C)], bd, sem)

    def wait(slot):
        b0, b1, bd = bufs[slot]
        sem = sems[slot]
        pltpu.make_async_copy(eflat_hbm.at[pl.ds(0, C)], b0, sem).wait()
        pltpu.make_async_copy(eflat_hbm.at[pl.ds(0, C)], b1, sem).wait()
        pltpu.make_async_copy(invd_hbm.at[pl.ds(0, C)], bd, sem).wait()

    def compute(slot, acc):
        b0, b1, bd = bufs[slot]

        @plsc.parallel_loop(0, C, step=L, unroll=4, carry=acc)
        def vec_body(o, acc):
            o = pl.multiple_of(o, L)
            idx0 = b0[pl.ds(o, L)]
            idx1 = b1[pl.ds(o, L)]
            p0 = plsc.load_gather(tab_v, [idx0])
            p1 = plsc.load_gather(tab_v, [idx1])
            w = bd[pl.ds(o, L)]
            x0 = plsc.bitcast(p0 << 16, jnp.float32)
            y0 = plsc.bitcast(p0 & _HI16, jnp.float32)
            x1 = plsc.bitcast(p1 << 16, jnp.float32)
            y1 = plsc.bitcast(p1 & _HI16, jnp.float32)
            dx = x0 - x1
            dy = y0 - y1
            s = dx * dx + dy * dy
            r = plsc.bitcast(_MAGIC - (plsc.bitcast(s, jnp.int32) >> 1),
                             jnp.float32)
            h = _CH * s
            r = r * (_C1 - h * r * r)
            eu = s * r
            q = eu * w - 1.0
            return acc + q * q

        return vec_body

    start(0, 0)
    tab_cp.wait()

    def outer(cc, acc):
        ci0 = cc * 2

        start(ci0 + 1, 1)
        wait(0)
        acc = compute(0, acc)

        @pl.when(cc + 1 < NCH // 2)
        def _():
            start(ci0 + 2, 0)

        wait(1)
        acc = compute(1, acc)
        return acc

    acc = lax.fori_loop(0, NCH // 2, outer, jnp.zeros((L,), jnp.float32))
    acc_v[...] = acc
    pltpu.sync_copy(acc_v, out_hbm.at[wid])


_sc_stress = pl.kernel(
    _sc_body,
    out_type=jax.ShapeDtypeStruct((NW, L), jnp.float32),
    mesh=plsc.VectorSubcoreMesh(
        core_axis_name="c", subcore_axis_name="s",
        num_cores=NC, num_subcores=NS),
    compiler_params=pltpu.CompilerParams(needs_layout_passes=False),
    scratch_types=[
        pltpu.VMEM((N,), jnp.int32),
        pltpu.VMEM((C,), jnp.int32),
        pltpu.VMEM((C,), jnp.int32),
        pltpu.VMEM((C,), jnp.float32),
        pltpu.VMEM((C,), jnp.int32),
        pltpu.VMEM((C,), jnp.int32),
        pltpu.VMEM((C,), jnp.float32),
        pltpu.VMEM((L,), jnp.float32),
        pltpu.SemaphoreType.DMA,
        pltpu.SemaphoreType.DMA,
        pltpu.SemaphoreType.DMA,
    ],
)


def kernel(node_pos, edge_index, edge_attr):
    nb = node_pos.astype(jnp.bfloat16)
    bits = lax.bitcast_convert_type(nb, jnp.uint16).astype(jnp.uint32)
    packed = lax.bitcast_convert_type(bits[:, 0] | (bits[:, 1] << 16),
                                      jnp.int32)
    eflat = edge_index.reshape(-1)
    inv_d = 1.0 / edge_attr[:, 0]
    partials = _sc_stress(packed, eflat, inv_d)
    return jnp.sum(partials)


# speedup vs baseline: 1.3657x; 1.0028x over previous
import jax
import jax.numpy as jnp
from jax import lax
from jax.experimental import pallas as pl
from jax.experimental.pallas import tpu as pltpu
from jax.experimental.pallas import tpu_sc as plsc

NC = 2
NS = 16
NW = NC * NS
L = 16

N = 100000
E = 6400000
EPW = E // NW
C = 4000
NCH = EPW // C

_MAGIC = 0x5F3759DF
_HI16 = -65536
_A = 1.00094
_C1 = 1.5 * _A
_CH = 0.5 * _A


def _sc_body(tab_hbm, eflat_hbm, invd_hbm, out_hbm,
             tab_v, i0a_v, i1a_v, da_v, i0b_v, i1b_v, db_v,
             acc_v, tsem, sem_a, sem_b):
    cid = lax.axis_index("c")
    sid = lax.axis_index("s")
    wid = sid * NC + cid
    base0 = wid * EPW

    tab_cp = pltpu.make_async_copy(tab_hbm, tab_v, tsem)
    tab_cp.start()

    sems = (sem_a, sem_b)
    bufs = ((i0a_v, i1a_v, da_v), (i0b_v, i1b_v, db_v))

    def start(ci, slot):
        base = pl.multiple_of(base0 + ci * C, 16)
        b0, b1, bd = bufs[slot]
        sem = sems[slot]
        pltpu.async_copy(eflat_hbm.at[pl.ds(base, C)], b0, sem)
        pltpu.async_copy(eflat_hbm.at[pl.ds(E + base, C)], b1, sem)
        pltpu.async_copy(invd_hbm.at[pl.ds(base, C)], bd, sem)

    def wait(slot):
        b0, b1, bd = bufs[slot]
        sem = sems[slot]
        pltpu.make_async_copy(eflat_hbm.at[pl.ds(0, C)], b0, sem).wait()
        pltpu.make_async_copy(eflat_hbm.at[pl.ds(0, C)], b1, sem).wait()
        pltpu.make_async_copy(invd_hbm.at[pl.ds(0, C)], bd, sem).wait()

    def compute(slot, acc):
        b0, b1, bd = bufs[slot]

        @plsc.parallel_loop(0, C, step=L, unroll=4, carry=acc)
        def vec_body(o, acc):
            o = pl.multiple_of(o, L)
            idx0 = b0[pl.ds(o, L)]
            idx1 = b1[pl.ds(o, L)]
            p0 = plsc.load_gather(tab_v, [idx0])
            p1 = plsc.load_gather(tab_v, [idx1])
            w = bd[pl.ds(o, L)]
            x0 = plsc.bitcast(p0 << 16, jnp.float32)
            y0 = plsc.bitcast(p0 & _HI16, jnp.float32)
            x1 = plsc.bitcast(p1 << 16, jnp.float32)
            y1 = plsc.bitcast(p1 & _HI16, jnp.float32)
            dx = x0 - x1
            dy = y0 - y1
            s = dx * dx + dy * dy
            r = plsc.bitcast(_MAGIC - (plsc.bitcast(s, jnp.int32) >> 1),
                             jnp.float32)
            h = _CH * s
            r = r * (_C1 - h * r * r)
            eu = s * r
            q = eu * w - 1.0
            return acc + q * q

        return vec_body

    start(0, 0)
    tab_cp.wait()

    def outer(cc, acc):
        ci0 = cc * 2

        start(ci0 + 1, 1)
        wait(0)
        acc = compute(0, acc)

        @pl.when(cc + 1 < NCH // 2)
        def _():
            start(ci0 + 2, 0)

        wait(1)
        acc = compute(1, acc)
        return acc

    acc = lax.fori_loop(0, NCH // 2, outer, jnp.zeros((L,), jnp.float32))
    acc_v[...] = acc
    pltpu.sync_copy(acc_v, out_hbm.at[wid])


_sc_stress = pl.kernel(
    _sc_body,
    out_type=jax.ShapeDtypeStruct((NW, L), jnp.float32),
    mesh=plsc.VectorSubcoreMesh(
        core_axis_name="c", subcore_axis_name="s",
        num_cores=NC, num_subcores=NS),
    compiler_params=pltpu.CompilerParams(needs_layout_passes=False),
    scratch_types=[
        pltpu.VMEM((N,), jnp.int32),
        pltpu.VMEM((C,), jnp.int32),
        pltpu.VMEM((C,), jnp.int32),
        pltpu.VMEM((C,), jnp.float32),
        pltpu.VMEM((C,), jnp.int32),
        pltpu.VMEM((C,), jnp.int32),
        pltpu.VMEM((C,), jnp.float32),
        pltpu.VMEM((L,), jnp.float32),
        pltpu.SemaphoreType.DMA,
        pltpu.SemaphoreType.DMA,
        pltpu.SemaphoreType.DMA,
    ],
)


def kernel(node_pos, edge_index, edge_attr):
    nb = node_pos.astype(jnp.bfloat16)
    bits = lax.bitcast_convert_type(nb, jnp.uint16).astype(jnp.uint32)
    packed = lax.bitcast_convert_type(bits[:, 0] | (bits[:, 1] << 16),
                                      jnp.int32)
    eflat = edge_index.reshape(-1)
    inv_d = 1.0 / edge_attr[:, 0]
    partials = _sc_stress(packed, eflat, inv_d)
    return jnp.sum(partials)
